# Initial kernel scaffold; baseline (speedup 1.0000x reference)
import jax
import jax.numpy as jnp
from jax.experimental import pallas as pl


def kernel(signal, hidden, episodic_keys, episodic_values, episodic_strength, episodic_age, W_key, b_key, W_val, b_val, W_write, b_write, W_pers, b_pers, W_merge, b_merge):
    raise NotImplementedError("write your pallas kernel here")


# speedup vs baseline: 1.5572x; 1.5572x over previous
import jax
import jax.numpy as jnp
from jax.experimental import pallas as pl
from jax.experimental.pallas import tpu as pltpu

STRENGTH_DECAY = 0.97
AGE_INCREMENT = 0.02
TEMPERATURE = 0.1

_ROWS = 16
_NEG = -1.0e30


def _first_argmax(x, ids, n):
    m = jnp.max(x, axis=-1, keepdims=True)
    idx = jnp.min(jnp.where(x == m, ids, n), axis=-1, keepdims=True)
    return m, idx


def _block_kernel(sig_ref, hid_ref, keys_ref, vals_ref, str_ref, age_ref,
                  wk_ref, bk_ref, wv_ref, bv_ref, ww_ref, bw_ref,
                  wp_ref, bp_ref, wm_ref, bm_ref,
                  ok_ref, ov_ref, os_ref, oa_ref):
    keys = keys_ref[...]
    vals = vals_ref[...]
    strength = str_ref[...]
    age = age_ref[...]
    R, N, D = keys.shape

    joined = jnp.concatenate([sig_ref[...], hid_ref[...]], axis=-1)
    ck = jnp.tanh(
        jax.lax.dot_general(joined, wk_ref[...], (((1,), (1,)), ((), ())),
                            preferred_element_type=jnp.float32) + bk_ref[...])
    cv = jnp.tanh(
        jax.lax.dot_general(joined, wv_ref[...], (((1,), (1,)), ((), ())),
                            preferred_element_type=jnp.float32) + bv_ref[...])
    write_strength = jax.nn.sigmoid(
        jnp.sum(joined * ww_ref[...], axis=-1, keepdims=True) + bw_ref[...])
    persistence = jax.nn.sigmoid(
        jnp.sum(joined * wp_ref[...], axis=-1, keepdims=True) + bp_ref[...])
    merge_logit = jnp.sum(joined * wm_ref[...], axis=-1, keepdims=True) + bm_ref[...]

    nc = ck / jnp.maximum(
        jnp.sqrt(jnp.sum(ck * ck, axis=-1, keepdims=True)), 1e-6)

    dot = jnp.sum(keys * nc[:, None, :], axis=-1)
    knorm = jnp.sqrt(jnp.sum(keys * keys, axis=-1))
    sim = dot / jnp.maximum(knorm, 1e-6)

    ids = jax.lax.broadcasted_iota(jnp.int32, (R, N), 1)

    m1, i1 = _first_argmax(sim, ids, N)
    sim_m = jnp.where(ids == i1, _NEG, sim)
    m2, i2 = _first_argmax(sim_m, ids, N)
    sim_m = jnp.where(ids == i2, _NEG, sim_m)
    m3, i3 = _first_argmax(sim_m, ids, N)

    replace_scores = 1.2 * age + (1.0 - strength) + 0.5 * (1.0 - sim)
    _, ri = _first_argmax(replace_scores, ids, N)

    novelty = jnp.clip(1.0 - m1, 0.0, 1.0)
    merge_pref = jax.nn.sigmoid(merge_logit + 2.6 * m1)
    full_m = (m1 > 0.78) & (merge_pref >= 0.55)
    multi_m = full_m & (m2 > 0.68)
    partial_m = (~multi_m) & (m1 > 0.64) & (m2 > 0.52)

    e2 = jnp.exp((m2 - m1) / TEMPERATURE)
    e3 = jnp.exp((m3 - m1) / TEMPERATURE)
    ps = 1.0 + e2
    ms = 1.0 + e2 + e3

    eq1 = (ids == i1).astype(jnp.float32)
    eq2 = (ids == i2).astype(jnp.float32)
    eq3 = (ids == i3).astype(jnp.float32)
    base_tw = (ids == ri).astype(jnp.float32)
    full_tw = eq1
    partial_tw = (eq1 + e2 * eq2) / ps
    multi_tw = (eq1 + e2 * eq2 + e3 * eq3) / ms

    tw = jnp.where(full_m, full_tw, base_tw)
    tw = jnp.where(partial_m, partial_tw, tw)
    tw = jnp.where(multi_m, multi_tw, tw)

    o_scale = jnp.where(multi_m, 0.16 + 0.52 * write_strength,
                        jnp.where(partial_m, 0.18 + 0.62 * write_strength,
                                  0.2 + 0.8 * write_strength))
    overwrite = tw * (o_scale * (0.55 + 0.45 * novelty))

    merge_like = full_m | partial_m | multi_m
    key_mix = jnp.where(merge_like, 0.28 + 0.24 * persistence,
                        0.78 + 0.16 * persistence)
    value_mix = jnp.where(merge_like, 0.42 + 0.28 * persistence,
                          0.82 + 0.12 * persistence)

    owk = (overwrite * key_mix)[..., None]
    owv = (overwrite * value_mix)[..., None]
    ok_ref[...] = keys + owk * (ck[:, None, :] - keys)
    ov_ref[...] = vals + owv * (cv[:, None, :] - vals)

    boost = overwrite * (0.45 + 0.35 * persistence
                         + 0.45 * novelty + 0.25 * write_strength)
    os_ref[...] = jnp.clip(strength * STRENGTH_DECAY + boost, 0.0, 1.0)
    oa_ref[...] = jnp.clip((age + AGE_INCREMENT) * (1.0 - overwrite), 0.0, 1.0)


def kernel(signal, hidden, episodic_keys, episodic_values, episodic_strength,
           episodic_age, W_key, b_key, W_val, b_val, W_write, b_write,
           W_pers, b_pers, W_merge, b_merge):
    B, N, D = episodic_keys.shape
    R = _ROWS
    grid = (B // R,)

    def row_blk(*shape_tail):
        return pl.BlockSpec((R,) + tuple(shape_tail), lambda i: (i,) + (0,) * len(shape_tail))

    def full_blk(shape):
        return pl.BlockSpec(shape, lambda i: (0,) * len(shape))

    b_key2 = b_key.reshape(1, D)
    b_val2 = b_val.reshape(1, D)
    b_write2 = b_write.reshape(1, 1)
    b_pers2 = b_pers.reshape(1, 1)
    b_merge2 = b_merge.reshape(1, 1)

    in_specs = [
        row_blk(D),
        row_blk(D),
        row_blk(N, D),
        row_blk(N, D),
        row_blk(N),
        row_blk(N),
        full_blk(W_key.shape),
        full_blk(b_key2.shape),
        full_blk(W_val.shape),
        full_blk(b_val2.shape),
        full_blk(W_write.shape),
        full_blk(b_write2.shape),
        full_blk(W_pers.shape),
        full_blk(b_pers2.shape),
        full_blk(W_merge.shape),
        full_blk(b_merge2.shape),
    ]
    out_specs = [
        row_blk(N, D),
        row_blk(N, D),
        row_blk(N),
        row_blk(N),
    ]
    out_shape = [
        jax.ShapeDtypeStruct((B, N, D), jnp.float32),
        jax.ShapeDtypeStruct((B, N, D), jnp.float32),
        jax.ShapeDtypeStruct((B, N), jnp.float32),
        jax.ShapeDtypeStruct((B, N), jnp.float32),
    ]

    out = pl.pallas_call(
        _block_kernel,
        grid=grid,
        in_specs=in_specs,
        out_specs=out_specs,
        out_shape=out_shape,
        compiler_params=pltpu.CompilerParams(
            dimension_semantics=("arbitrary",),
        ),
    )(signal, hidden, episodic_keys, episodic_values, episodic_strength,
      episodic_age, W_key, b_key2, W_val, b_val2, W_write, b_write2,
      W_pers, b_pers2, W_merge, b_merge2)
    return tuple(out)


# speedup vs baseline: 1.9488x; 1.2515x over previous
import jax
import jax.numpy as jnp
from jax.experimental import pallas as pl
from jax.experimental.pallas import tpu as pltpu

STRENGTH_DECAY = 0.97
AGE_INCREMENT = 0.02
TEMPERATURE = 0.1

_ROWS = 16
_NEG = -1.0e30


def _first_argmax(x, ids, n):
    m = jnp.max(x, axis=-1, keepdims=True)
    idx = jnp.min(jnp.where(x == m, ids, n), axis=-1, keepdims=True)
    return m, idx


def _block_kernel(sig_ref, hid_ref, keys_ref, vals_ref, str_ref, age_ref,
                  wk_ref, bk_ref, wv_ref, bv_ref, ww_ref, bw_ref,
                  wp_ref, bp_ref, wm_ref, bm_ref,
                  ok_ref, ov_ref, os_ref, oa_ref):
    keys = keys_ref[...]
    vals = vals_ref[...]
    strength = str_ref[...]
    age = age_ref[...]
    R, N, D = keys.shape

    joined = jnp.concatenate([sig_ref[...], hid_ref[...]], axis=-1)
    ck = jnp.tanh(
        jax.lax.dot_general(joined, wk_ref[...], (((1,), (1,)), ((), ())),
                            preferred_element_type=jnp.float32) + bk_ref[...])
    cv = jnp.tanh(
        jax.lax.dot_general(joined, wv_ref[...], (((1,), (1,)), ((), ())),
                            preferred_element_type=jnp.float32) + bv_ref[...])
    write_strength = jax.nn.sigmoid(
        jnp.sum(joined * ww_ref[...], axis=-1, keepdims=True) + bw_ref[...])
    persistence = jax.nn.sigmoid(
        jnp.sum(joined * wp_ref[...], axis=-1, keepdims=True) + bp_ref[...])
    merge_logit = jnp.sum(joined * wm_ref[...], axis=-1, keepdims=True) + bm_ref[...]

    nc = ck / jnp.maximum(
        jnp.sqrt(jnp.sum(ck * ck, axis=-1, keepdims=True)), 1e-6)

    dot = jnp.sum(keys * nc[:, None, :], axis=-1)
    knorm = jnp.sqrt(jnp.sum(keys * keys, axis=-1))
    sim = dot / jnp.maximum(knorm, 1e-6)

    ids = jax.lax.broadcasted_iota(jnp.int32, (R, N), 1)

    m1, i1 = _first_argmax(sim, ids, N)
    sim_m = jnp.where(ids == i1, _NEG, sim)
    m2, i2 = _first_argmax(sim_m, ids, N)
    sim_m = jnp.where(ids == i2, _NEG, sim_m)
    m3, i3 = _first_argmax(sim_m, ids, N)

    replace_scores = 1.2 * age + (1.0 - strength) + 0.5 * (1.0 - sim)
    _, ri = _first_argmax(replace_scores, ids, N)

    novelty = jnp.clip(1.0 - m1, 0.0, 1.0)
    merge_pref = jax.nn.sigmoid(merge_logit + 2.6 * m1)
    full_m = (m1 > 0.78) & (merge_pref >= 0.55)
    multi_m = full_m & (m2 > 0.68)
    partial_m = (~multi_m) & (m1 > 0.64) & (m2 > 0.52)

    e2 = jnp.exp((m2 - m1) / TEMPERATURE)
    e3 = jnp.exp((m3 - m1) / TEMPERATURE)
    ps = 1.0 + e2
    ms = 1.0 + e2 + e3

    eq1 = (ids == i1).astype(jnp.float32)
    eq2 = (ids == i2).astype(jnp.float32)
    eq3 = (ids == i3).astype(jnp.float32)
    base_tw = (ids == ri).astype(jnp.float32)
    full_tw = eq1
    partial_tw = (eq1 + e2 * eq2) / ps
    multi_tw = (eq1 + e2 * eq2 + e3 * eq3) / ms

    tw = jnp.where(full_m, full_tw, base_tw)
    tw = jnp.where(partial_m, partial_tw, tw)
    tw = jnp.where(multi_m, multi_tw, tw)

    o_scale = jnp.where(multi_m, 0.16 + 0.52 * write_strength,
                        jnp.where(partial_m, 0.18 + 0.62 * write_strength,
                                  0.2 + 0.8 * write_strength))
    overwrite = tw * (o_scale * (0.55 + 0.45 * novelty))

    merge_like = full_m | partial_m | multi_m
    key_mix = jnp.where(merge_like, 0.28 + 0.24 * persistence,
                        0.78 + 0.16 * persistence)
    value_mix = jnp.where(merge_like, 0.42 + 0.28 * persistence,
                          0.82 + 0.12 * persistence)

    boost = overwrite * (0.45 + 0.35 * persistence
                         + 0.45 * novelty + 0.25 * write_strength)
    os_ref[...] = jnp.clip(strength * STRENGTH_DECAY + boost, 0.0, 1.0)
    oa_ref[...] = jnp.clip((age + AGE_INCREMENT) * (1.0 - overwrite), 0.0, 1.0)

    ok_ref[...] = keys
    ov_ref[...] = vals
    scale = o_scale * (0.55 + 0.45 * novelty)
    w0 = jnp.where(multi_m, 1.0 / ms,
                   jnp.where(partial_m, 1.0 / ps, 1.0))
    w1 = jnp.where(multi_m, e2 / ms,
                   jnp.where(partial_m, e2 / ps, 0.0))
    w2 = jnp.where(multi_m, e3 / ms, 0.0)
    t0 = jnp.where(merge_like, i1, ri)
    kc0 = w0 * scale * key_mix
    kc1 = w1 * scale * key_mix
    kc2 = w2 * scale * key_mix
    vc0 = w0 * scale * value_mix
    vc1 = w1 * scale * value_mix
    vc2 = w2 * scale * value_mix
    for r in range(R):
        ck_row = ck[r:r + 1, :]
        cv_row = cv[r:r + 1, :]
        for tj, kc, vc in ((i3, kc2, vc2), (i2, kc1, vc1), (t0, kc0, vc0)):
            tt = tj[r, 0]
            krow = keys_ref[r, pl.ds(tt, 1), :]
            ok_ref[r, pl.ds(tt, 1), :] = krow + kc[r:r + 1, 0:1] * (ck_row - krow)
            vrow = vals_ref[r, pl.ds(tt, 1), :]
            ov_ref[r, pl.ds(tt, 1), :] = vrow + vc[r:r + 1, 0:1] * (cv_row - vrow)


def kernel(signal, hidden, episodic_keys, episodic_values, episodic_strength,
           episodic_age, W_key, b_key, W_val, b_val, W_write, b_write,
           W_pers, b_pers, W_merge, b_merge):
    B, N, D = episodic_keys.shape
    R = _ROWS
    grid = (B // R,)

    def row_blk(*shape_tail):
        return pl.BlockSpec((R,) + tuple(shape_tail), lambda i: (i,) + (0,) * len(shape_tail))

    def full_blk(shape):
        return pl.BlockSpec(shape, lambda i: (0,) * len(shape))

    b_key2 = b_key.reshape(1, D)
    b_val2 = b_val.reshape(1, D)
    b_write2 = b_write.reshape(1, 1)
    b_pers2 = b_pers.reshape(1, 1)
    b_merge2 = b_merge.reshape(1, 1)

    in_specs = [
        row_blk(D),
        row_blk(D),
        row_blk(N, D),
        row_blk(N, D),
        row_blk(N),
        row_blk(N),
        full_blk(W_key.shape),
        full_blk(b_key2.shape),
        full_blk(W_val.shape),
        full_blk(b_val2.shape),
        full_blk(W_write.shape),
        full_blk(b_write2.shape),
        full_blk(W_pers.shape),
        full_blk(b_pers2.shape),
        full_blk(W_merge.shape),
        full_blk(b_merge2.shape),
    ]
    out_specs = [
        row_blk(N, D),
        row_blk(N, D),
        row_blk(N),
        row_blk(N),
    ]
    out_shape = [
        jax.ShapeDtypeStruct((B, N, D), jnp.float32),
        jax.ShapeDtypeStruct((B, N, D), jnp.float32),
        jax.ShapeDtypeStruct((B, N), jnp.float32),
        jax.ShapeDtypeStruct((B, N), jnp.float32),
    ]

    out = pl.pallas_call(
        _block_kernel,
        grid=grid,
        in_specs=in_specs,
        out_specs=out_specs,
        out_shape=out_shape,
        compiler_params=pltpu.CompilerParams(
            dimension_semantics=("arbitrary",),
        ),
    )(signal, hidden, episodic_keys, episodic_values, episodic_strength,
      episodic_age, W_key, b_key2, W_val, b_val2, W_write, b_write2,
      W_pers, b_pers2, W_merge, b_merge2)
    return tuple(out)


# speedup vs baseline: 2.2827x; 1.1713x over previous
import jax
import jax.numpy as jnp
from jax.experimental import pallas as pl
from jax.experimental.pallas import tpu as pltpu

STRENGTH_DECAY = 0.97
AGE_INCREMENT = 0.02
TEMPERATURE = 0.1

_ROWS = 16
_NEG = -1.0e30


def _first_argmax(x, ids, n):
    m = jnp.max(x, axis=-1, keepdims=True)
    idx = jnp.argmax(x, axis=-1)[:, None].astype(jnp.int32)
    return m, idx


def _block_kernel(sig_ref, hid_ref, keys_ref, vals_ref, str_ref, age_ref,
                  wk_ref, bk_ref, wv_ref, bv_ref, ww_ref, bw_ref,
                  wp_ref, bp_ref, wm_ref, bm_ref,
                  ok_ref, ov_ref, os_ref, oa_ref):
    keys = keys_ref[...]
    vals = vals_ref[...]
    strength = str_ref[...]
    age = age_ref[...]
    R, N, D = keys.shape

    joined = jnp.concatenate([sig_ref[...], hid_ref[...]], axis=-1)
    ck = jnp.tanh(
        jax.lax.dot_general(joined, wk_ref[...], (((1,), (1,)), ((), ())),
                            preferred_element_type=jnp.float32) + bk_ref[...])
    cv = jnp.tanh(
        jax.lax.dot_general(joined, wv_ref[...], (((1,), (1,)), ((), ())),
                            preferred_element_type=jnp.float32) + bv_ref[...])
    write_strength = jax.nn.sigmoid(
        jnp.sum(joined * ww_ref[...], axis=-1, keepdims=True) + bw_ref[...])
    persistence = jax.nn.sigmoid(
        jnp.sum(joined * wp_ref[...], axis=-1, keepdims=True) + bp_ref[...])
    merge_logit = jnp.sum(joined * wm_ref[...], axis=-1, keepdims=True) + bm_ref[...]

    nc = ck / jnp.maximum(
        jnp.sqrt(jnp.sum(ck * ck, axis=-1, keepdims=True)), 1e-6)

    dot = jnp.sum(keys * nc[:, None, :], axis=-1)
    kn2 = jnp.sum(keys * keys, axis=-1)
    sim = dot * jax.lax.rsqrt(jnp.maximum(kn2, 1e-12))

    ids = jax.lax.broadcasted_iota(jnp.int32, (R, N), 1)

    m1, i1 = _first_argmax(sim, ids, N)
    sim_m = jnp.where(ids == i1, _NEG, sim)
    m2, i2 = _first_argmax(sim_m, ids, N)
    sim_m = jnp.where(ids == i2, _NEG, sim_m)
    m3, i3 = _first_argmax(sim_m, ids, N)

    replace_scores = 1.2 * age + (1.0 - strength) + 0.5 * (1.0 - sim)
    _, ri = _first_argmax(replace_scores, ids, N)

    novelty = jnp.clip(1.0 - m1, 0.0, 1.0)
    merge_pref = jax.nn.sigmoid(merge_logit + 2.6 * m1)
    full_m = (m1 > 0.78) & (merge_pref >= 0.55)
    multi_m = full_m & (m2 > 0.68)
    partial_m = (~multi_m) & (m1 > 0.64) & (m2 > 0.52)

    e2 = jnp.exp((m2 - m1) / TEMPERATURE)
    e3 = jnp.exp((m3 - m1) / TEMPERATURE)
    ps = 1.0 + e2
    ms = 1.0 + e2 + e3

    eq1 = (ids == i1).astype(jnp.float32)
    eq2 = (ids == i2).astype(jnp.float32)
    eq3 = (ids == i3).astype(jnp.float32)
    base_tw = (ids == ri).astype(jnp.float32)
    full_tw = eq1
    partial_tw = (eq1 + e2 * eq2) / ps
    multi_tw = (eq1 + e2 * eq2 + e3 * eq3) / ms

    tw = jnp.where(full_m, full_tw, base_tw)
    tw = jnp.where(partial_m, partial_tw, tw)
    tw = jnp.where(multi_m, multi_tw, tw)

    o_scale = jnp.where(multi_m, 0.16 + 0.52 * write_strength,
                        jnp.where(partial_m, 0.18 + 0.62 * write_strength,
                                  0.2 + 0.8 * write_strength))
    overwrite = tw * (o_scale * (0.55 + 0.45 * novelty))

    merge_like = full_m | partial_m | multi_m
    key_mix = jnp.where(merge_like, 0.28 + 0.24 * persistence,
                        0.78 + 0.16 * persistence)
    value_mix = jnp.where(merge_like, 0.42 + 0.28 * persistence,
                          0.82 + 0.12 * persistence)

    boost = overwrite * (0.45 + 0.35 * persistence
                         + 0.45 * novelty + 0.25 * write_strength)
    os_ref[...] = jnp.clip(strength * STRENGTH_DECAY + boost, 0.0, 1.0)
    oa_ref[...] = jnp.clip((age + AGE_INCREMENT) * (1.0 - overwrite), 0.0, 1.0)

    ok_ref[...] = keys
    ov_ref[...] = vals
    scale = o_scale * (0.55 + 0.45 * novelty)
    w0 = jnp.where(multi_m, 1.0 / ms,
                   jnp.where(partial_m, 1.0 / ps, 1.0))
    w1 = jnp.where(multi_m, e2 / ms,
                   jnp.where(partial_m, e2 / ps, 0.0))
    w2 = jnp.where(multi_m, e3 / ms, 0.0)
    t0 = jnp.where(merge_like, i1, ri)
    kc0 = w0 * scale * key_mix
    kc1 = w1 * scale * key_mix
    kc2 = w2 * scale * key_mix
    vc0 = w0 * scale * value_mix
    vc1 = w1 * scale * value_mix
    vc2 = w2 * scale * value_mix
    for r in range(R):
        ck_row = ck[r:r + 1, :]
        cv_row = cv[r:r + 1, :]
        for tj, kc, vc in ((i3, kc2, vc2), (i2, kc1, vc1), (t0, kc0, vc0)):
            tt = tj[r, 0]
            krow = keys_ref[r, pl.ds(tt, 1), :]
            ok_ref[r, pl.ds(tt, 1), :] = krow + kc[r:r + 1, 0:1] * (ck_row - krow)
            vrow = vals_ref[r, pl.ds(tt, 1), :]
            ov_ref[r, pl.ds(tt, 1), :] = vrow + vc[r:r + 1, 0:1] * (cv_row - vrow)


def kernel(signal, hidden, episodic_keys, episodic_values, episodic_strength,
           episodic_age, W_key, b_key, W_val, b_val, W_write, b_write,
           W_pers, b_pers, W_merge, b_merge):
    B, N, D = episodic_keys.shape
    R = _ROWS
    grid = (B // R,)

    def row_blk(*shape_tail):
        return pl.BlockSpec((R,) + tuple(shape_tail), lambda i: (i,) + (0,) * len(shape_tail))

    def full_blk(shape):
        return pl.BlockSpec(shape, lambda i: (0,) * len(shape))

    b_key2 = b_key.reshape(1, D)
    b_val2 = b_val.reshape(1, D)
    b_write2 = b_write.reshape(1, 1)
    b_pers2 = b_pers.reshape(1, 1)
    b_merge2 = b_merge.reshape(1, 1)

    in_specs = [
        row_blk(D),
        row_blk(D),
        row_blk(N, D),
        row_blk(N, D),
        row_blk(N),
        row_blk(N),
        full_blk(W_key.shape),
        full_blk(b_key2.shape),
        full_blk(W_val.shape),
        full_blk(b_val2.shape),
        full_blk(W_write.shape),
        full_blk(b_write2.shape),
        full_blk(W_pers.shape),
        full_blk(b_pers2.shape),
        full_blk(W_merge.shape),
        full_blk(b_merge2.shape),
    ]
    out_specs = [
        row_blk(N, D),
        row_blk(N, D),
        row_blk(N),
        row_blk(N),
    ]
    out_shape = [
        jax.ShapeDtypeStruct((B, N, D), jnp.float32),
        jax.ShapeDtypeStruct((B, N, D), jnp.float32),
        jax.ShapeDtypeStruct((B, N), jnp.float32),
        jax.ShapeDtypeStruct((B, N), jnp.float32),
    ]

    out = pl.pallas_call(
        _block_kernel,
        grid=grid,
        in_specs=in_specs,
        out_specs=out_specs,
        out_shape=out_shape,
        compiler_params=pltpu.CompilerParams(
            dimension_semantics=("arbitrary",),
        ),
    )(signal, hidden, episodic_keys, episodic_values, episodic_strength,
      episodic_age, W_key, b_key2, W_val, b_val2, W_write, b_write2,
      W_pers, b_pers2, W_merge, b_merge2)
    return tuple(out)
